# baseline (speedup 1.0000x reference)
import functools

import jax
import jax.numpy as jnp
from jax import lax
from jax.experimental import pallas as pl
from jax.experimental.pallas import tpu as pltpu
from jax.experimental.pallas import tpu_sc as plsc

D = 256
NJ = D // 16
B = 64
NPART = 4
CH = 2048
_EPS = 1e-16


def _round_up(x, m):
    return (x + m - 1) // m * m


def _mm_body(x_ref, w_ref, b_ref, o_ref):
    o_ref[...] = (
        jnp.dot(x_ref[...], w_ref[...], preferred_element_type=jnp.float32)
        + b_ref[...]
    )


def _project(x_pad, w_cat, b_cat, interpret=False):
    n, k = x_pad.shape
    m = w_cat.shape[1]
    bn, bm = 256, 512
    return pl.pallas_call(
        _mm_body,
        grid=(n // bn, m // bm),
        in_specs=[
            pl.BlockSpec((bn, k), lambda i, j: (i, 0)),
            pl.BlockSpec((k, bm), lambda i, j: (0, j)),
            pl.BlockSpec((1, bm), lambda i, j: (0, j)),
        ],
        out_specs=pl.BlockSpec((bn, bm), lambda i, j: (i, j)),
        out_shape=jax.ShapeDtypeStruct((n, m), jnp.float32),
        compiler_params=pltpu.CompilerParams(
            dimension_semantics=("parallel", "parallel")
        ),
        interpret=interpret,
    )(x_pad, w_cat, b_cat)


def _sc_conv_body(n_dst, e_pad, n_ea, interpret,
                  xl_hbm, xr_hbm, s_hbm, d_hbm, att_hbm, wea_hbm, ea_hbm,
                  out_hbm,
                  spm, ks, kd, keas, s_ch, d_ch, ea_chs, xl_buf, xr_buf,
                  out_buf, widx, didx, att_v, wea_v):
    q = n_dst // NPART
    qpad = _round_up(q + 1, 128)
    rpt = qpad // 16
    sh = e_pad // 16

    cid = lax.axis_index("c")
    sid = lax.axis_index("sub")
    r0 = sid * rpt
    shard_lo = sid * sh

    pltpu.sync_copy(att_hbm, att_v)
    if n_ea:
        pltpu.sync_copy(wea_hbm, wea_v)

    zv = jnp.zeros((16,), jnp.float32)

    def _zero_row(i, _):
        for j in range(NJ + 1):
            out_buf[i, pl.ds(16 * j, 16)] = zv
        return 0

    pad_d = jnp.full((16,), q, jnp.int32)
    pad_s = jnp.zeros((16,), jnp.int32)
    pad_f = jnp.zeros((16,), jnp.float32)

    def _pass(p, _):
        part = 2 * cid + p
        lo = part * q

        lax.fori_loop(0, B, _zero_row, 0)
        for c0 in range(0, rpt, B):
            sz = min(B, rpt - c0)
            pltpu.sync_copy(out_buf.at[pl.ds(0, sz)],
                            spm.at[pl.ds(r0 + c0, sz)])
        plsc.subcore_barrier()

        lo_v = jnp.full((16,), lo, jnp.int32)

        def _batch(b, _):
            off = b * B
            for t in range(B // 16):
                sl = pl.ds(16 * t, 16)
                widx[sl] = kd[pl.ds(off + 16 * t, 16)]
                didx[sl] = kd[pl.ds(off + 16 * t, 16)] + lo_v
            pltpu.sync_copy(xr_hbm.at[didx], xr_buf)
            for t in range(B // 16):
                didx[pl.ds(16 * t, 16)] = ks[pl.ds(off + 16 * t, 16)]
            pltpu.sync_copy(xl_hbm.at[didx], xl_buf)

            def _edge(i, _):
                acc = jnp.zeros((16,), jnp.float32)
                bidx = jnp.full((16,), off + i, jnp.int32)
                eav = [plsc.load_gather(keas[t], [bidx]) for t in range(n_ea)]
                for j in range(NJ):
                    sl = pl.ds(16 * j, 16)
                    m = xl_buf[i, sl] + xr_buf[i, sl]
                    for t in range(n_ea):
                        m = m + eav[t] * wea_v[t, j]
                    g = jnp.maximum(m, 0.2 * m)
                    acc = acc + g * att_v[j]
                ex = jnp.exp(jnp.full((16,), jnp.sum(acc), jnp.float32))
                for j in range(NJ):
                    sl = pl.ds(16 * j, 16)
                    out_buf[i, sl] = ex * xl_buf[i, sl]
                out_buf[i, pl.ds(D, 16)] = ex
                return 0

            lax.fori_loop(0, B, _edge, 0)
            pltpu.sync_copy(out_buf, spm.at[widx], add=True)
            return 0

        def _chunk_body(c0, sz):
            pltpu.sync_copy(s_hbm.at[pl.ds(c0, sz)], s_ch.at[pl.ds(0, sz)])
            pltpu.sync_copy(d_hbm.at[pl.ds(c0, sz)], d_ch.at[pl.ds(0, sz)])
            for t in range(n_ea):
                pltpu.sync_copy(ea_hbm.at[t, pl.ds(c0, sz)],
                                ea_chs[t].at[pl.ds(0, sz)])

            def _step(k, cnt):
                dv = d_ch[pl.ds(16 * k, 16)]
                sv = s_ch[pl.ds(16 * k, 16)]
                msk = (dv >= lo) & (dv < lo + q)
                plsc.store_compressed(ks.at[pl.ds(cnt, 16)], sv, mask=msk)
                plsc.store_compressed(kd.at[pl.ds(cnt, 16)], dv - lo, mask=msk)
                for t in range(n_ea):
                    plsc.store_compressed(keas[t].at[pl.ds(cnt, 16)],
                                          ea_chs[t][pl.ds(16 * k, 16)],
                                          mask=msk)
                return cnt + plsc.all_reduce_population_count(msk)[0]

            cnt = lax.fori_loop(0, sz // 16, _step, 0)

            for t in range(B // 16):
                ks[pl.ds(cnt + 16 * t, 16)] = pad_s
                kd[pl.ds(cnt + 16 * t, 16)] = pad_d
                for u in range(n_ea):
                    keas[u][pl.ds(cnt + 16 * t, 16)] = pad_f
            nb = (cnt + B - 1) // B
            lax.fori_loop(0, nb, _batch, 0)

        nfull = sh // CH
        rem = sh % CH
        if nfull:
            def _chunk(c, _):
                _chunk_body(shard_lo + c * CH, CH)
                return 0
            lax.fori_loop(0, nfull, _chunk, 0)
        if rem:
            _chunk_body(shard_lo + nfull * CH, rem)
        plsc.subcore_barrier()

        pltpu.sync_copy(spm.at[pl.ds(r0, rpt)],
                        out_hbm.at[pl.ds(part * qpad + r0, rpt)])
        return 0

    lax.fori_loop(0, 2, _pass, 0)


def _sc_conv(xl, xr, s_idx, d_idx, att, wea, ea_t, n_dst, interpret=False):
    e_pad = s_idx.shape[0]
    n_ea = 0 if ea_t is None else ea_t.shape[0]
    q = n_dst // NPART
    qpad = _round_up(q + 1, 128)
    sh = e_pad // 16
    ch = min(sh, CH)
    cap = CH + 2 * B

    if ea_t is None:
        ea_t = jnp.zeros((1, e_pad), jnp.float32)
        wea = jnp.zeros((1, 16, 16), jnp.float32)

    mesh = plsc.VectorSubcoreMesh(core_axis_name="c", subcore_axis_name="sub",
                                  num_cores=2, num_subcores=16)
    scratch = [
        pltpu.VMEM_SHARED((qpad, D + 16), jnp.float32),
        pltpu.VMEM((cap,), jnp.int32),
        pltpu.VMEM((cap,), jnp.int32),
        [pltpu.VMEM((cap,), jnp.float32) for _ in range(n_ea)],
        pltpu.VMEM((ch,), jnp.int32),
        pltpu.VMEM((ch,), jnp.int32),
        [pltpu.VMEM((ch,), jnp.float32) for _ in range(n_ea)],
        pltpu.VMEM((B, D), jnp.float32),
        pltpu.VMEM((B, D), jnp.float32),
        pltpu.VMEM((B, D + 16), jnp.float32),
        pltpu.VMEM((B,), jnp.int32),
        pltpu.VMEM((B,), jnp.int32),
        pltpu.VMEM((NJ, 16), jnp.float32),
        pltpu.VMEM((max(n_ea, 1), NJ, 16), jnp.float32),
    ]
    body = functools.partial(_sc_conv_body, n_dst, e_pad, n_ea, interpret)
    fn = pl.kernel(
        body,
        out_type=jax.ShapeDtypeStruct((NPART * qpad, D + 16), jnp.float32),
        mesh=mesh,
        scratch_types=scratch,
        compiler_params=pltpu.CompilerParams(use_tc_tiling_on_sc=False,
                                             needs_layout_passes=False),
        interpret=interpret,
    )
    return fn(xl, xr, s_idx, d_idx, att, wea, ea_t)


def _make_combine_body(n_conv):
    def _body(*refs):
        x_ref = refs[0]
        b_ref = refs[1]
        o_ref = refs[-1]
        acc = x_ref[...] + b_ref[...]
        for c in range(n_conv):
            nd = refs[2 + c][...]
            acc = acc + nd[:, :D] / (nd[:, D:D + 1] + _EPS)
        o_ref[...] = acc
    return _body


def _combine(x_pad, bias_sum, numdens, interpret=False):
    n = x_pad.shape[0]
    br = 256
    n_conv = len(numdens)
    return pl.pallas_call(
        _make_combine_body(n_conv),
        grid=(n // br,),
        in_specs=[
            pl.BlockSpec((br, D), lambda i: (i, 0)),
            pl.BlockSpec((1, D), lambda i: (0, 0)),
        ] + [pl.BlockSpec((br, D + 16), lambda i: (i, 0)) for _ in numdens],
        out_specs=pl.BlockSpec((br, D), lambda i: (i, 0)),
        out_shape=jax.ShapeDtypeStruct((n, D), jnp.float32),
        compiler_params=pltpu.CompilerParams(
            dimension_semantics=("parallel",)
        ),
        interpret=interpret,
    )(x_pad, bias_sum, *numdens)


def _pad_rows(x, n):
    return jnp.pad(x, ((0, n - x.shape[0]), (0, 0)))


def _pad_edges(ei, ea, n_dst):
    e = ei.shape[1]
    e_pad = _round_up(e, 256)
    s = jnp.pad(ei[0], (0, e_pad - e)).astype(jnp.int32)
    d = jnp.pad(ei[1], (0, e_pad - e),
                constant_values=2 * n_dst + 64).astype(jnp.int32)
    if ea is None:
        ea_t = None
    else:
        ea_t = jnp.pad(ea.T, ((0, 0), (0, e_pad - e))).astype(jnp.float32)
    return s, d, ea_t


def kernel(x_operation, x_machine, x_AGV, ei_op_pred, ei_op_succ,
           ei_m_processable, ei_m_processing, ea_m_processing,
           ei_m_waiting, ea_m_waiting, ei_a_pos_m, ei_a_tgt_m,
           ea_a_tgt_m, ei_a_pos_o, ei_a_tgt_o, params, interpret=False):
    p = params
    n_op, n_m, n_a = x_operation.shape[0], x_machine.shape[0], x_AGV.shape[0]
    n_op_pad, n_m_pad, n_a_pad = (_round_up(n_op, 256), _round_up(n_m, 256),
                                  _round_up(n_a, 256))
    x_op_pad = _pad_rows(x_operation, n_op_pad)
    x_m_pad = _pad_rows(x_machine, n_m_pad)
    x_a_pad = _pad_rows(x_AGV, n_a_pad)

    convs = [
        ("op_pred", p["op_pred"], "op", "op", ei_op_pred, False, None),
        ("op_succ", p["op_succ"], "op", "op", ei_op_succ, False, None),
        ("m_proc_f", p["m_processable"]["fwd"], "m", "op", ei_m_processable, False, None),
        ("m_proc_b", p["m_processable"]["bwd"], "op", "m", ei_m_processable, True, None),
        ("m_ing_f", p["m_processing"]["fwd"], "m", "op", ei_m_processing, False, ea_m_processing),
        ("m_ing_b", p["m_processing"]["bwd"], "op", "m", ei_m_processing, True, ea_m_processing),
        ("m_wait_f", p["m_waiting"]["fwd"], "m", "op", ei_m_waiting, False, ea_m_waiting),
        ("m_wait_b", p["m_waiting"]["bwd"], "op", "m", ei_m_waiting, True, ea_m_waiting),
        ("a_pos_m_f", p["a_pos_m"]["fwd"], "a", "m", ei_a_pos_m, False, None),
        ("a_pos_m_b", p["a_pos_m"]["bwd"], "m", "a", ei_a_pos_m, True, None),
        ("a_tgt_m_f", p["a_tgt_m"]["fwd"], "a", "m", ei_a_tgt_m, False, ea_a_tgt_m),
        ("a_tgt_m_b", p["a_tgt_m"]["bwd"], "m", "a", ei_a_tgt_m, True, ea_a_tgt_m),
        ("a_pos_o_f", p["a_pos_o"]["fwd"], "a", "op", ei_a_pos_o, False, None),
        ("a_pos_o_b", p["a_pos_o"]["bwd"], "op", "a", ei_a_pos_o, True, None),
        ("a_tgt_o_f", p["a_tgt_o"]["fwd"], "a", "op", ei_a_tgt_o, False, None),
        ("a_tgt_o_b", p["a_tgt_o"]["bwd"], "op", "a", ei_a_tgt_o, True, None),
    ]

    mats = {"op": [], "m": [], "a": []}
    for ci, (_, cp, sk, dk, _, _, _) in enumerate(convs):
        mats[sk].append((cp["Wl"], cp["bl"], ci, "l"))
        mats[dk].append((cp["Wr"], cp["br"], ci, "r"))

    xs = {"op": x_op_pad, "m": x_m_pad, "a": x_a_pad}
    tables = {}
    for kind, lst in mats.items():
        w_cat = jnp.concatenate([w for w, _, _, _ in lst], axis=1)
        b_cat = jnp.concatenate([b for _, b, _, _ in lst])[None, :]
        proj = _project(xs[kind], w_cat, b_cat, interpret=interpret)
        for t, (_, _, ci, side) in enumerate(lst):
            tables[(ci, side)] = proj[:, t * D:(t + 1) * D]

    n_dst_of = {"op": n_op, "m": n_m, "a": n_a}

    numdens = {"op": [], "m": [], "a": []}
    biases = {"op": [], "m": [], "a": []}
    for ci, (_, cp, sk, dk, ei, swap, ea) in enumerate(convs):
        n_dst = n_dst_of[dk]
        s_row, d_row = (ei[1], ei[0]) if swap else (ei[0], ei[1])
        s_idx, d_idx, ea_t = _pad_edges(jnp.stack([s_row, d_row]), ea, n_dst)
        att = cp["att"].reshape(NJ, 16)
        wea = (cp["We"].reshape(-1, NJ, 16) if ea is not None else None)
        nd = _sc_conv(tables[(ci, "l")], tables[(ci, "r")], s_idx, d_idx,
                      att, wea, ea_t, n_dst, interpret=interpret)
        q = n_dst // NPART
        qpad = _round_up(q + 1, 128)
        nd_real = jnp.concatenate(
            [nd[i * qpad:i * qpad + q] for i in range(NPART)], axis=0)
        n_pad = {"op": n_op_pad, "m": n_m_pad, "a": n_a_pad}[dk]
        numdens[dk].append(jnp.pad(nd_real, ((0, n_pad - n_dst), (0, 0))))
        biases[dk].append(cp["bias"])

    res = {}
    for kind, x_pad in (("op", x_op_pad), ("m", x_m_pad), ("a", x_a_pad)):
        bias_sum = sum(biases[kind])[None, :]
        res[kind] = _combine(x_pad, bias_sum, numdens[kind], interpret=interpret)

    return jnp.concatenate(
        [res["op"][:n_op], res["m"][:n_m], res["a"][:n_a]], axis=0)


# speedup vs baseline: 1.0797x; 1.0797x over previous
import functools

import jax
import jax.numpy as jnp
from jax import lax
from jax.experimental import pallas as pl
from jax.experimental.pallas import tpu as pltpu
from jax.experimental.pallas import tpu_sc as plsc

D = 256
NJ = D // 16
B = 64
NPART = 4
CH = 2048
_EPS = 1e-16


def _round_up(x, m):
    return (x + m - 1) // m * m


def _mm_body(x_ref, w_ref, b_ref, o_ref):
    o_ref[...] = (
        jnp.dot(x_ref[...], w_ref[...], preferred_element_type=jnp.float32)
        + b_ref[...]
    )


def _project(x_pad, w_cat, b_cat, interpret=False):
    n, k = x_pad.shape
    m = w_cat.shape[1]
    bn, bm = 256, 512
    return pl.pallas_call(
        _mm_body,
        grid=(n // bn, m // bm),
        in_specs=[
            pl.BlockSpec((bn, k), lambda i, j: (i, 0)),
            pl.BlockSpec((k, bm), lambda i, j: (0, j)),
            pl.BlockSpec((1, bm), lambda i, j: (0, j)),
        ],
        out_specs=pl.BlockSpec((bn, bm), lambda i, j: (i, j)),
        out_shape=jax.ShapeDtypeStruct((n, m), jnp.float32),
        compiler_params=pltpu.CompilerParams(
            dimension_semantics=("parallel", "parallel")
        ),
        interpret=interpret,
    )(x_pad, w_cat, b_cat)


def _sc_conv_body(n_dst, e_pad, n_ea, interpret,
                  xl_hbm, xr_hbm, s_hbm, d_hbm, att_hbm, wea_hbm, ea_hbm,
                  out_hbm,
                  spm, ks, kd, keas, s_ch, d_ch, ea_chs, xl_buf, xr_buf,
                  out_buf, widx, didx, sidx, att_v, wea_v, sem_r, sem_l):
    q = n_dst // NPART
    qpad = _round_up(q + 1, 128)
    rpt = qpad // 16
    sh = e_pad // 16

    cid = lax.axis_index("c")
    sid = lax.axis_index("sub")
    r0 = sid * rpt
    shard_lo = sid * sh

    pltpu.sync_copy(att_hbm, att_v)
    if n_ea:
        pltpu.sync_copy(wea_hbm, wea_v)

    zv = jnp.zeros((16,), jnp.float32)

    def _zero_row(i, _):
        for j in range(NJ + 1):
            out_buf[i, pl.ds(16 * j, 16)] = zv
        return 0

    pad_d = jnp.full((16,), q, jnp.int32)
    pad_s = jnp.zeros((16,), jnp.int32)
    pad_f = jnp.zeros((16,), jnp.float32)

    def _pass(p, _):
        part = 2 * cid + p
        lo = part * q

        lax.fori_loop(0, B, _zero_row, 0)
        for c0 in range(0, rpt, B):
            sz = min(B, rpt - c0)
            pltpu.sync_copy(out_buf.at[pl.ds(0, sz)],
                            spm.at[pl.ds(r0 + c0, sz)])
        plsc.subcore_barrier()

        lo_v = jnp.full((16,), lo, jnp.int32)

        def _batch(b, _):
            off = b * B
            for t in range(B // 16):
                sl = pl.ds(16 * t, 16)
                kdv = kd[pl.ds(off + 16 * t, 16)]
                widx[sl] = kdv
                didx[sl] = kdv + lo_v
                sidx[sl] = ks[pl.ds(off + 16 * t, 16)]
            cpr = pltpu.async_copy(xr_hbm.at[didx], xr_buf, sem_r)
            cpl = pltpu.async_copy(xl_hbm.at[sidx], xl_buf, sem_l)
            cpr.wait()
            cpl.wait()

            def _edge(i, _):
                acc = jnp.zeros((16,), jnp.float32)
                bidx = jnp.full((16,), off + i, jnp.int32)
                eav = [plsc.load_gather(keas[t], [bidx]) for t in range(n_ea)]
                for j in range(NJ):
                    sl = pl.ds(16 * j, 16)
                    m = xl_buf[i, sl] + xr_buf[i, sl]
                    for t in range(n_ea):
                        m = m + eav[t] * wea_v[t, j]
                    g = jnp.maximum(m, 0.2 * m)
                    acc = acc + g * att_v[j]
                ex = jnp.exp(jnp.full((16,), jnp.sum(acc), jnp.float32))
                for j in range(NJ):
                    sl = pl.ds(16 * j, 16)
                    out_buf[i, sl] = ex * xl_buf[i, sl]
                out_buf[i, pl.ds(D, 16)] = ex
                return 0

            lax.fori_loop(0, B, _edge, 0)
            pltpu.sync_copy(out_buf, spm.at[widx], add=True)
            return 0

        def _chunk_body(c0, sz):
            pltpu.sync_copy(s_hbm.at[pl.ds(c0, sz)], s_ch.at[pl.ds(0, sz)])
            pltpu.sync_copy(d_hbm.at[pl.ds(c0, sz)], d_ch.at[pl.ds(0, sz)])
            for t in range(n_ea):
                pltpu.sync_copy(ea_hbm.at[t, pl.ds(c0, sz)],
                                ea_chs[t].at[pl.ds(0, sz)])

            def _step(k, cnt):
                dv = d_ch[pl.ds(16 * k, 16)]
                sv = s_ch[pl.ds(16 * k, 16)]
                msk = (dv >= lo) & (dv < lo + q)
                plsc.store_compressed(ks.at[pl.ds(cnt, 16)], sv, mask=msk)
                plsc.store_compressed(kd.at[pl.ds(cnt, 16)], dv - lo, mask=msk)
                for t in range(n_ea):
                    plsc.store_compressed(keas[t].at[pl.ds(cnt, 16)],
                                          ea_chs[t][pl.ds(16 * k, 16)],
                                          mask=msk)
                return cnt + plsc.all_reduce_population_count(msk)[0]

            cnt = lax.fori_loop(0, sz // 16, _step, 0)

            for t in range(B // 16):
                ks[pl.ds(cnt + 16 * t, 16)] = pad_s
                kd[pl.ds(cnt + 16 * t, 16)] = pad_d
                for u in range(n_ea):
                    keas[u][pl.ds(cnt + 16 * t, 16)] = pad_f
            nb = (cnt + B - 1) // B
            lax.fori_loop(0, nb, _batch, 0)

        nfull = sh // CH
        rem = sh % CH
        if nfull:
            def _chunk(c, _):
                _chunk_body(shard_lo + c * CH, CH)
                return 0
            lax.fori_loop(0, nfull, _chunk, 0)
        if rem:
            _chunk_body(shard_lo + nfull * CH, rem)
        plsc.subcore_barrier()

        pltpu.sync_copy(spm.at[pl.ds(r0, rpt)],
                        out_hbm.at[pl.ds(part * qpad + r0, rpt)])
        return 0

    lax.fori_loop(0, 2, _pass, 0)


def _sc_conv(xl, xr, s_idx, d_idx, att, wea, ea_t, n_dst, interpret=False):
    e_pad = s_idx.shape[0]
    n_ea = 0 if ea_t is None else ea_t.shape[0]
    q = n_dst // NPART
    qpad = _round_up(q + 1, 128)
    sh = e_pad // 16
    ch = min(sh, CH)
    cap = CH + 2 * B

    if ea_t is None:
        ea_t = jnp.zeros((1, e_pad), jnp.float32)
        wea = jnp.zeros((1, 16, 16), jnp.float32)

    mesh = plsc.VectorSubcoreMesh(core_axis_name="c", subcore_axis_name="sub",
                                  num_cores=2, num_subcores=16)
    scratch = [
        pltpu.VMEM_SHARED((qpad, D + 16), jnp.float32),
        pltpu.VMEM((cap,), jnp.int32),
        pltpu.VMEM((cap,), jnp.int32),
        [pltpu.VMEM((cap,), jnp.float32) for _ in range(n_ea)],
        pltpu.VMEM((ch,), jnp.int32),
        pltpu.VMEM((ch,), jnp.int32),
        [pltpu.VMEM((ch,), jnp.float32) for _ in range(n_ea)],
        pltpu.VMEM((B, D), jnp.float32),
        pltpu.VMEM((B, D), jnp.float32),
        pltpu.VMEM((B, D + 16), jnp.float32),
        pltpu.VMEM((B,), jnp.int32),
        pltpu.VMEM((B,), jnp.int32),
        pltpu.VMEM((B,), jnp.int32),
        pltpu.VMEM((NJ, 16), jnp.float32),
        pltpu.VMEM((max(n_ea, 1), NJ, 16), jnp.float32),
        pltpu.SemaphoreType.DMA,
        pltpu.SemaphoreType.DMA,
    ]
    body = functools.partial(_sc_conv_body, n_dst, e_pad, n_ea, interpret)
    fn = pl.kernel(
        body,
        out_type=jax.ShapeDtypeStruct((NPART * qpad, D + 16), jnp.float32),
        mesh=mesh,
        scratch_types=scratch,
        compiler_params=pltpu.CompilerParams(use_tc_tiling_on_sc=False,
                                             needs_layout_passes=False),
        interpret=interpret,
    )
    return fn(xl, xr, s_idx, d_idx, att, wea, ea_t)


def _make_combine_body(n_conv):
    def _body(*refs):
        x_ref = refs[0]
        b_ref = refs[1]
        o_ref = refs[-1]
        acc = x_ref[...] + b_ref[...]
        for c in range(n_conv):
            nd = refs[2 + c][...]
            acc = acc + nd[:, :D] / (nd[:, D:D + 1] + _EPS)
        o_ref[...] = acc
    return _body


def _combine(x_pad, bias_sum, numdens, interpret=False):
    n = x_pad.shape[0]
    br = 256
    n_conv = len(numdens)
    return pl.pallas_call(
        _make_combine_body(n_conv),
        grid=(n // br,),
        in_specs=[
            pl.BlockSpec((br, D), lambda i: (i, 0)),
            pl.BlockSpec((1, D), lambda i: (0, 0)),
        ] + [pl.BlockSpec((br, D + 16), lambda i: (i, 0)) for _ in numdens],
        out_specs=pl.BlockSpec((br, D), lambda i: (i, 0)),
        out_shape=jax.ShapeDtypeStruct((n, D), jnp.float32),
        compiler_params=pltpu.CompilerParams(
            dimension_semantics=("parallel",)
        ),
        interpret=interpret,
    )(x_pad, bias_sum, *numdens)


def _pad_rows(x, n):
    return jnp.pad(x, ((0, n - x.shape[0]), (0, 0)))


def _pad_edges(ei, ea, n_dst):
    e = ei.shape[1]
    e_pad = _round_up(e, 256)
    s = jnp.pad(ei[0], (0, e_pad - e)).astype(jnp.int32)
    d = jnp.pad(ei[1], (0, e_pad - e),
                constant_values=2 * n_dst + 64).astype(jnp.int32)
    if ea is None:
        ea_t = None
    else:
        ea_t = jnp.pad(ea.T, ((0, 0), (0, e_pad - e))).astype(jnp.float32)
    return s, d, ea_t


def kernel(x_operation, x_machine, x_AGV, ei_op_pred, ei_op_succ,
           ei_m_processable, ei_m_processing, ea_m_processing,
           ei_m_waiting, ea_m_waiting, ei_a_pos_m, ei_a_tgt_m,
           ea_a_tgt_m, ei_a_pos_o, ei_a_tgt_o, params, interpret=False):
    p = params
    n_op, n_m, n_a = x_operation.shape[0], x_machine.shape[0], x_AGV.shape[0]
    n_op_pad, n_m_pad, n_a_pad = (_round_up(n_op, 256), _round_up(n_m, 256),
                                  _round_up(n_a, 256))
    x_op_pad = _pad_rows(x_operation, n_op_pad)
    x_m_pad = _pad_rows(x_machine, n_m_pad)
    x_a_pad = _pad_rows(x_AGV, n_a_pad)

    convs = [
        ("op_pred", p["op_pred"], "op", "op", ei_op_pred, False, None),
        ("op_succ", p["op_succ"], "op", "op", ei_op_succ, False, None),
        ("m_proc_f", p["m_processable"]["fwd"], "m", "op", ei_m_processable, False, None),
        ("m_proc_b", p["m_processable"]["bwd"], "op", "m", ei_m_processable, True, None),
        ("m_ing_f", p["m_processing"]["fwd"], "m", "op", ei_m_processing, False, ea_m_processing),
        ("m_ing_b", p["m_processing"]["bwd"], "op", "m", ei_m_processing, True, ea_m_processing),
        ("m_wait_f", p["m_waiting"]["fwd"], "m", "op", ei_m_waiting, False, ea_m_waiting),
        ("m_wait_b", p["m_waiting"]["bwd"], "op", "m", ei_m_waiting, True, ea_m_waiting),
        ("a_pos_m_f", p["a_pos_m"]["fwd"], "a", "m", ei_a_pos_m, False, None),
        ("a_pos_m_b", p["a_pos_m"]["bwd"], "m", "a", ei_a_pos_m, True, None),
        ("a_tgt_m_f", p["a_tgt_m"]["fwd"], "a", "m", ei_a_tgt_m, False, ea_a_tgt_m),
        ("a_tgt_m_b", p["a_tgt_m"]["bwd"], "m", "a", ei_a_tgt_m, True, ea_a_tgt_m),
        ("a_pos_o_f", p["a_pos_o"]["fwd"], "a", "op", ei_a_pos_o, False, None),
        ("a_pos_o_b", p["a_pos_o"]["bwd"], "op", "a", ei_a_pos_o, True, None),
        ("a_tgt_o_f", p["a_tgt_o"]["fwd"], "a", "op", ei_a_tgt_o, False, None),
        ("a_tgt_o_b", p["a_tgt_o"]["bwd"], "op", "a", ei_a_tgt_o, True, None),
    ]

    mats = {"op": [], "m": [], "a": []}
    for ci, (_, cp, sk, dk, _, _, _) in enumerate(convs):
        mats[sk].append((cp["Wl"], cp["bl"], ci, "l"))
        mats[dk].append((cp["Wr"], cp["br"], ci, "r"))

    xs = {"op": x_op_pad, "m": x_m_pad, "a": x_a_pad}
    tables = {}
    for kind, lst in mats.items():
        w_cat = jnp.concatenate([w for w, _, _, _ in lst], axis=1)
        b_cat = jnp.concatenate([b for _, b, _, _ in lst])[None, :]
        proj = _project(xs[kind], w_cat, b_cat, interpret=interpret)
        for t, (_, _, ci, side) in enumerate(lst):
            tables[(ci, side)] = proj[:, t * D:(t + 1) * D]

    n_dst_of = {"op": n_op, "m": n_m, "a": n_a}

    numdens = {"op": [], "m": [], "a": []}
    biases = {"op": [], "m": [], "a": []}
    for ci, (_, cp, sk, dk, ei, swap, ea) in enumerate(convs):
        n_dst = n_dst_of[dk]
        s_row, d_row = (ei[1], ei[0]) if swap else (ei[0], ei[1])
        s_idx, d_idx, ea_t = _pad_edges(jnp.stack([s_row, d_row]), ea, n_dst)
        att = cp["att"].reshape(NJ, 16)
        wea = (cp["We"].reshape(-1, NJ, 16) if ea is not None else None)
        nd = _sc_conv(tables[(ci, "l")], tables[(ci, "r")], s_idx, d_idx,
                      att, wea, ea_t, n_dst, interpret=interpret)
        q = n_dst // NPART
        qpad = _round_up(q + 1, 128)
        nd_real = jnp.concatenate(
            [nd[i * qpad:i * qpad + q] for i in range(NPART)], axis=0)
        n_pad = {"op": n_op_pad, "m": n_m_pad, "a": n_a_pad}[dk]
        numdens[dk].append(jnp.pad(nd_real, ((0, n_pad - n_dst), (0, 0))))
        biases[dk].append(cp["bias"])

    res = {}
    for kind, x_pad in (("op", x_op_pad), ("m", x_m_pad), ("a", x_a_pad)):
        bias_sum = sum(biases[kind])[None, :]
        res[kind] = _combine(x_pad, bias_sum, numdens[kind], interpret=interpret)

    return jnp.concatenate(
        [res["op"][:n_op], res["m"][:n_m], res["a"][:n_a]], axis=0)


# speedup vs baseline: 1.6686x; 1.5455x over previous
import functools

import jax
import jax.numpy as jnp
from jax import lax
from jax.experimental import pallas as pl
from jax.experimental.pallas import tpu as pltpu
from jax.experimental.pallas import tpu_sc as plsc

D = 256
NJ = D // 16
B = 64
NPART = 4
NPART_BIG = 8
CH = 1024
_EPS = 1e-16


def _round_up(x, m):
    return (x + m - 1) // m * m


def _mm_body(x_ref, w_ref, b_ref, o_ref):
    o_ref[...] = (
        jnp.dot(x_ref[...], w_ref[...], preferred_element_type=jnp.float32)
        + b_ref[...]
    )


def _project(x_pad, w_cat, b_cat, interpret=False):
    n, k = x_pad.shape
    m = w_cat.shape[1]
    bn, bm = 256, 512
    return pl.pallas_call(
        _mm_body,
        grid=(n // bn, m // bm),
        in_specs=[
            pl.BlockSpec((bn, k), lambda i, j: (i, 0)),
            pl.BlockSpec((k, bm), lambda i, j: (0, j)),
            pl.BlockSpec((1, bm), lambda i, j: (0, j)),
        ],
        out_specs=pl.BlockSpec((bn, bm), lambda i, j: (i, j)),
        out_shape=jax.ShapeDtypeStruct((n, m), jnp.float32),
        compiler_params=pltpu.CompilerParams(
            dimension_semantics=("parallel", "parallel")
        ),
        interpret=interpret,
    )(x_pad, w_cat, b_cat)


def _sc_conv_body(n_dst, e_pad, n_ea, npart, bsz, interpret,
                  xl_hbm, xr_hbm, s_hbm, d_hbm, att_hbm, wea_hbm, ea_hbm,
                  out_hbm,
                  spm, ks, kd, keas, s_ch, d_ch, ea_chs, xl_b, xr_b,
                  out_b, widx, didx, sidx, att_v, wea_v, abuf, exv,
                  sem_r, sem_l, sem_s):
    q = n_dst // npart
    qpad = _round_up(q + 1, 128)
    rpt = qpad // 16
    sh = e_pad // 16
    nch = sh // CH
    npass = npart // 2
    bq = bsz // 16

    cid = lax.axis_index("c")
    sid = lax.axis_index("sub")
    r0 = sid * rpt
    shard_lo = sid * sh

    pltpu.sync_copy(att_hbm, att_v)
    if n_ea:
        pltpu.sync_copy(wea_hbm, wea_v)

    zv = jnp.zeros((16,), jnp.float32)

    def _zero_row(i, _):
        for j in range(NJ + 1):
            out_b[0][i, pl.ds(16 * j, 16)] = zv
        return 0

    pad_d = jnp.full((16,), q, jnp.int32)
    pad_s = jnp.zeros((16,), jnp.int32)
    pad_f = jnp.zeros((16,), jnp.float32)

    def _pass(p, _):
        part = npass * cid + p
        lo = part * q

        lax.fori_loop(0, bsz, _zero_row, 0)
        for c0 in range(0, rpt, bsz):
            sz = min(bsz, rpt - c0)
            pltpu.sync_copy(out_b[0].at[pl.ds(0, sz)],
                            spm.at[pl.ds(r0 + c0, sz)])
        plsc.subcore_barrier()

        lo_v = jnp.full((16,), lo, jnp.int32)

        def _issue(off, par, wr):
            for t in range(bq):
                sl = pl.ds(16 * t, 16)
                kdv = kd[pl.ds(off + 16 * t, 16)]
                widx[wr][sl] = kdv
                didx[par][sl] = kdv + lo_v
                sidx[par][sl] = ks[pl.ds(off + 16 * t, 16)]
            pltpu.async_copy(xr_hbm.at[didx[par]], xr_b[par], sem_r[par])
            pltpu.async_copy(xl_hbm.at[sidx[par]], xl_b[par], sem_l[par])

        def _gather_wait(par):
            pltpu.make_async_copy(xr_hbm.at[didx[par]], xr_b[par],
                                  sem_r[par]).wait()
            pltpu.make_async_copy(xl_hbm.at[sidx[par]], xl_b[par],
                                  sem_l[par]).wait()

        def _scatter_wait(wr, sp):
            pltpu.make_async_copy(out_b[sp], spm.at[widx[wr]],
                                  sem_s[sp]).wait()

        def _compute(off, par, wr):
            @plsc.parallel_loop(0, bsz, unroll=4)
            def _edge_a(i):
                bidx = jnp.full((16,), off + i, jnp.int32)
                eav = [plsc.load_gather(keas[t], [bidx]) for t in range(n_ea)]
                accs = [jnp.zeros((16,), jnp.float32) for _ in range(4)]
                for j in range(NJ):
                    sl = pl.ds(16 * j, 16)
                    m = xl_b[par][i, sl] + xr_b[par][i, sl]
                    for t in range(n_ea):
                        m = m + eav[t] * wea_v[t, j]
                    g = jnp.maximum(m, 0.2 * m)
                    accs[j % 4] = accs[j % 4] + g * att_v[j]
                abuf[pl.ds(16 * i, 16)] = (accs[0] + accs[1]) + (accs[2]
                                                                 + accs[3])

            iota16 = lax.iota(jnp.int32, 16) * 16

            @plsc.parallel_loop(0, bsz // 16, unroll=1)
            def _group_b(gidx):
                base = iota16 + gidx * 256
                tots = [plsc.load_gather(abuf, [base + j]) for j in range(4)]
                for j in range(4, 16):
                    tots[j % 4] = tots[j % 4] + plsc.load_gather(
                        abuf, [base + j])
                exv[pl.ds(16 * gidx, 16)] = jnp.exp((tots[0] + tots[1])
                                                    + (tots[2] + tots[3]))

            @plsc.parallel_loop(0, bsz, unroll=4)
            def _edge_c(i):
                ex = plsc.load_gather(exv, [jnp.full((16,), i, jnp.int32)])
                for j in range(NJ):
                    sl = pl.ds(16 * j, 16)
                    out_b[par][i, sl] = ex * xl_b[par][i, sl]
                out_b[par][i, pl.ds(D, 16)] = ex

            pltpu.async_copy(out_b[par], spm.at[widx[wr]], sem_s[par],
                             add=True)

        def _chunk(c, carry):
            gb, rem = carry
            c0 = shard_lo + c * CH
            pltpu.sync_copy(s_hbm.at[pl.ds(c0, CH)], s_ch)
            pltpu.sync_copy(d_hbm.at[pl.ds(c0, CH)], d_ch)
            for t in range(n_ea):
                pltpu.sync_copy(ea_hbm.at[t, pl.ds(c0, CH)], ea_chs[t])

            def _step(k, cnt):
                dv = d_ch[pl.ds(16 * k, 16)]
                sv = s_ch[pl.ds(16 * k, 16)]
                msk = (dv >= lo) & (dv < lo + q)
                plsc.store_compressed(ks.at[pl.ds(cnt, 16)], sv, mask=msk)
                plsc.store_compressed(kd.at[pl.ds(cnt, 16)], dv - lo,
                                      mask=msk)
                for t in range(n_ea):
                    plsc.store_compressed(keas[t].at[pl.ds(cnt, 16)],
                                          ea_chs[t][pl.ds(16 * k, 16)],
                                          mask=msk)
                return cnt + plsc.all_reduce_population_count(msk)[0]

            cnt = lax.fori_loop(0, CH // 16, _step, rem)

            def _finish(cnt):
                for t in range(bq):
                    ks[pl.ds(cnt + 16 * t, 16)] = pad_s
                    kd[pl.ds(cnt + 16 * t, 16)] = pad_d
                    for u in range(n_ea):
                        keas[u][pl.ds(cnt + 16 * t, 16)] = pad_f

            nb = cnt // bsz
            if True:
                is_last = c == nch - 1
                nb = jnp.where(is_last, (cnt + bsz - 1) // bsz, nb)

                @pl.when(jnp.logical_and(is_last, cnt % bsz != 0))
                def _():
                    _finish(cnt)

            def _pipe_step(bi, _):
                g = gb + bi

                def _one(k):
                    @pl.when(bi + 1 < nb)
                    def _():
                        _issue((bi + 1) * bsz, (k + 1) % 2, (k + 1) % 4)

                    @pl.when(g >= 2)
                    def _():
                        _scatter_wait((k + 2) % 4, k % 2)

                    _gather_wait(k % 2)
                    _compute(bi * bsz, k % 2, k)

                for k in range(4):
                    @pl.when(g % 4 == k)
                    def _(k=k):
                        _one(k)

                return 0

            @pl.when(nb >= 1)
            def _():
                for k in range(4):
                    @pl.when(gb % 4 == k)
                    def _(k=k):
                        _issue(0, k % 2, k)

            lax.fori_loop(0, nb, _pipe_step, 0)

            new_rem = cnt - nb * bsz

            @pl.when(jnp.logical_and(nb >= 1, new_rem > 0))
            def _():
                for t in range(bq):
                    sl = pl.ds(16 * t, 16)
                    ks[sl] = ks[pl.ds(nb * bsz + 16 * t, 16)]
                    kd[sl] = kd[pl.ds(nb * bsz + 16 * t, 16)]
                    for u in range(n_ea):
                        keas[u][sl] = keas[u][pl.ds(nb * bsz + 16 * t, 16)]

            return gb + nb, new_rem

        total, _rem = lax.fori_loop(0, nch, _chunk, (0, 0))

        for k in range(4):
            @pl.when(jnp.logical_and(total >= 1, (total - 1) % 4 == k))
            def _(k=k):
                _scatter_wait(k, k % 2)

            @pl.when(jnp.logical_and(total >= 2, (total - 2) % 4 == k))
            def _(k=k):
                _scatter_wait(k, k % 2)

        plsc.subcore_barrier()
        pltpu.sync_copy(spm.at[pl.ds(r0, rpt)],
                        out_hbm.at[pl.ds(cid * npass * qpad + p * qpad + r0,
                                         rpt)])
        return 0

    lax.fori_loop(0, npass, _pass, 0)


def _sc_conv(xl, xr, s_idx, d_idx, att, wea, ea_t, n_dst, interpret=False):
    e_pad = s_idx.shape[0]
    n_ea = 0 if ea_t is None else ea_t.shape[0]
    npart = NPART_BIG if n_dst >= 4096 else NPART
    bsz = 48 if n_ea else B
    q = n_dst // npart
    qpad = _round_up(q + 1, 128)
    cap = CH + 2 * B

    if ea_t is None:
        ea_t = jnp.zeros((1, e_pad), jnp.float32)
        wea = jnp.zeros((1, 16, 16), jnp.float32)

    mesh = plsc.VectorSubcoreMesh(core_axis_name="c", subcore_axis_name="sub",
                                  num_cores=2, num_subcores=16)
    scratch = [
        pltpu.VMEM_SHARED((qpad, D + 16), jnp.float32),
        pltpu.VMEM((cap,), jnp.int32),
        pltpu.VMEM((cap,), jnp.int32),
        [pltpu.VMEM((cap,), jnp.float32) for _ in range(n_ea)],
        pltpu.VMEM((CH,), jnp.int32),
        pltpu.VMEM((CH,), jnp.int32),
        [pltpu.VMEM((CH,), jnp.float32) for _ in range(n_ea)],
        [pltpu.VMEM((bsz, D), jnp.float32) for _ in range(2)],
        [pltpu.VMEM((bsz, D), jnp.float32) for _ in range(2)],
        [pltpu.VMEM((bsz, D + 16), jnp.float32) for _ in range(2)],
        [pltpu.VMEM((bsz,), jnp.int32) for _ in range(4)],
        [pltpu.VMEM((bsz,), jnp.int32) for _ in range(2)],
        [pltpu.VMEM((bsz,), jnp.int32) for _ in range(2)],
        pltpu.VMEM((NJ, 16), jnp.float32),
        pltpu.VMEM((max(n_ea, 1), NJ, 16), jnp.float32),
        pltpu.VMEM((bsz * 16,), jnp.float32),
        pltpu.VMEM((bsz,), jnp.float32),
        [pltpu.SemaphoreType.DMA for _ in range(2)],
        [pltpu.SemaphoreType.DMA for _ in range(2)],
        [pltpu.SemaphoreType.DMA for _ in range(2)],
    ]
    body = functools.partial(_sc_conv_body, n_dst, e_pad, n_ea, npart, bsz,
                             interpret)
    fn = pl.kernel(
        body,
        out_type=jax.ShapeDtypeStruct((npart * qpad, D + 16), jnp.float32),
        mesh=mesh,
        scratch_types=scratch,
        compiler_params=pltpu.CompilerParams(use_tc_tiling_on_sc=False,
                                             needs_layout_passes=False),
        interpret=interpret,
    )
    return fn(xl, xr, s_idx, d_idx, att, wea, ea_t)


def _make_combine_body(n_conv):
    def _body(*refs):
        x_ref = refs[0]
        b_ref = refs[1]
        o_ref = refs[-1]
        acc = x_ref[...] + b_ref[...]
        for c in range(n_conv):
            nd = refs[2 + c][...]
            acc = acc + nd[:, :D] / (nd[:, D:D + 1] + _EPS)
        o_ref[...] = acc
    return _body


def _combine(x_pad, bias_sum, numdens, interpret=False):
    n = x_pad.shape[0]
    br = 256
    n_conv = len(numdens)
    return pl.pallas_call(
        _make_combine_body(n_conv),
        grid=(n // br,),
        in_specs=[
            pl.BlockSpec((br, D), lambda i: (i, 0)),
            pl.BlockSpec((1, D), lambda i: (0, 0)),
        ] + [pl.BlockSpec((br, D + 16), lambda i: (i, 0)) for _ in numdens],
        out_specs=pl.BlockSpec((br, D), lambda i: (i, 0)),
        out_shape=jax.ShapeDtypeStruct((n, D), jnp.float32),
        compiler_params=pltpu.CompilerParams(
            dimension_semantics=("parallel",)
        ),
        interpret=interpret,
    )(x_pad, bias_sum, *numdens)


def _pad_rows(x, n):
    return jnp.pad(x, ((0, n - x.shape[0]), (0, 0)))


def _pad_edges(ei, ea, n_dst):
    e = ei.shape[1]
    e_pad = _round_up(e, 16 * 1024)
    s = jnp.pad(ei[0], (0, e_pad - e)).astype(jnp.int32)
    d = jnp.pad(ei[1], (0, e_pad - e),
                constant_values=2 * n_dst + 64).astype(jnp.int32)
    if ea is None:
        ea_t = None
    else:
        ea_t = jnp.pad(ea.T, ((0, 0), (0, e_pad - e))).astype(jnp.float32)
    return s, d, ea_t


def kernel(x_operation, x_machine, x_AGV, ei_op_pred, ei_op_succ,
           ei_m_processable, ei_m_processing, ea_m_processing,
           ei_m_waiting, ea_m_waiting, ei_a_pos_m, ei_a_tgt_m,
           ea_a_tgt_m, ei_a_pos_o, ei_a_tgt_o, params, interpret=False):
    p = params
    n_op, n_m, n_a = x_operation.shape[0], x_machine.shape[0], x_AGV.shape[0]
    n_op_pad, n_m_pad, n_a_pad = (_round_up(n_op, 256), _round_up(n_m, 256),
                                  _round_up(n_a, 256))
    x_op_pad = _pad_rows(x_operation, n_op_pad)
    x_m_pad = _pad_rows(x_machine, n_m_pad)
    x_a_pad = _pad_rows(x_AGV, n_a_pad)

    convs = [
        ("op_pred", p["op_pred"], "op", "op", ei_op_pred, False, None),
        ("op_succ", p["op_succ"], "op", "op", ei_op_succ, False, None),
        ("m_proc_f", p["m_processable"]["fwd"], "m", "op", ei_m_processable, False, None),
        ("m_proc_b", p["m_processable"]["bwd"], "op", "m", ei_m_processable, True, None),
        ("m_ing_f", p["m_processing"]["fwd"], "m", "op", ei_m_processing, False, ea_m_processing),
        ("m_ing_b", p["m_processing"]["bwd"], "op", "m", ei_m_processing, True, ea_m_processing),
        ("m_wait_f", p["m_waiting"]["fwd"], "m", "op", ei_m_waiting, False, ea_m_waiting),
        ("m_wait_b", p["m_waiting"]["bwd"], "op", "m", ei_m_waiting, True, ea_m_waiting),
        ("a_pos_m_f", p["a_pos_m"]["fwd"], "a", "m", ei_a_pos_m, False, None),
        ("a_pos_m_b", p["a_pos_m"]["bwd"], "m", "a", ei_a_pos_m, True, None),
        ("a_tgt_m_f", p["a_tgt_m"]["fwd"], "a", "m", ei_a_tgt_m, False, ea_a_tgt_m),
        ("a_tgt_m_b", p["a_tgt_m"]["bwd"], "m", "a", ei_a_tgt_m, True, ea_a_tgt_m),
        ("a_pos_o_f", p["a_pos_o"]["fwd"], "a", "op", ei_a_pos_o, False, None),
        ("a_pos_o_b", p["a_pos_o"]["bwd"], "op", "a", ei_a_pos_o, True, None),
        ("a_tgt_o_f", p["a_tgt_o"]["fwd"], "a", "op", ei_a_tgt_o, False, None),
        ("a_tgt_o_b", p["a_tgt_o"]["bwd"], "op", "a", ei_a_tgt_o, True, None),
    ]

    mats = {"op": [], "m": [], "a": []}
    for ci, (_, cp, sk, dk, _, _, _) in enumerate(convs):
        mats[sk].append((cp["Wl"], cp["bl"], ci, "l"))
        mats[dk].append((cp["Wr"], cp["br"], ci, "r"))

    xs = {"op": x_op_pad, "m": x_m_pad, "a": x_a_pad}
    tables = {}
    for kind, lst in mats.items():
        w_cat = jnp.concatenate([w for w, _, _, _ in lst], axis=1)
        b_cat = jnp.concatenate([b for _, b, _, _ in lst])[None, :]
        proj = _project(xs[kind], w_cat, b_cat, interpret=interpret)
        for t, (_, _, ci, side) in enumerate(lst):
            tables[(ci, side)] = proj[:, t * D:(t + 1) * D]

    n_dst_of = {"op": n_op, "m": n_m, "a": n_a}

    numdens = {"op": [], "m": [], "a": []}
    biases = {"op": [], "m": [], "a": []}
    for ci, (_, cp, sk, dk, ei, swap, ea) in enumerate(convs):
        n_dst = n_dst_of[dk]
        s_row, d_row = (ei[1], ei[0]) if swap else (ei[0], ei[1])
        s_idx, d_idx, ea_t = _pad_edges(jnp.stack([s_row, d_row]), ea, n_dst)
        att = cp["att"].reshape(NJ, 16)
        wea = (cp["We"].reshape(-1, NJ, 16) if ea is not None else None)
        nd = _sc_conv(tables[(ci, "l")], tables[(ci, "r")], s_idx, d_idx,
                      att, wea, ea_t, n_dst, interpret=interpret)
        npart = NPART_BIG if n_dst >= 4096 else NPART
        q = n_dst // npart
        qpad = _round_up(q + 1, 128)
        nd_real = jnp.concatenate(
            [nd[i * qpad:i * qpad + q] for i in range(npart)], axis=0)
        n_pad = {"op": n_op_pad, "m": n_m_pad, "a": n_a_pad}[dk]
        numdens[dk].append(jnp.pad(nd_real, ((0, n_pad - n_dst), (0, 0))))
        biases[dk].append(cp["bias"])

    res = {}
    for kind, x_pad in (("op", x_op_pad), ("m", x_m_pad), ("a", x_a_pad)):
        bias_sum = sum(biases[kind])[None, :]
        res[kind] = _combine(x_pad, bias_sum, numdens[kind], interpret=interpret)

    return jnp.concatenate(
        [res["op"][:n_op], res["m"][:n_m], res["a"][:n_a]], axis=0)


# speedup vs baseline: 1.7096x; 1.0246x over previous
import functools

import jax
import jax.numpy as jnp
from jax import lax
from jax.experimental import pallas as pl
from jax.experimental.pallas import tpu as pltpu
from jax.experimental.pallas import tpu_sc as plsc

D = 256
NJ = D // 16
B = 64
NPART = 4
NPART_BIG = 8
CH = 1024
_EPS = 1e-16


def _round_up(x, m):
    return (x + m - 1) // m * m


def _mm_body(x_ref, w_ref, b_ref, o_ref):
    o_ref[...] = (
        jnp.dot(x_ref[...], w_ref[...], preferred_element_type=jnp.float32)
        + b_ref[...]
    )


def _project(x_pad, w_cat, b_cat, interpret=False):
    n, k = x_pad.shape
    m = w_cat.shape[1]
    bn, bm = 256, 512
    return pl.pallas_call(
        _mm_body,
        grid=(n // bn, m // bm),
        in_specs=[
            pl.BlockSpec((bn, k), lambda i, j: (i, 0)),
            pl.BlockSpec((k, bm), lambda i, j: (0, j)),
            pl.BlockSpec((1, bm), lambda i, j: (0, j)),
        ],
        out_specs=pl.BlockSpec((bn, bm), lambda i, j: (i, j)),
        out_shape=jax.ShapeDtypeStruct((n, m), jnp.float32),
        compiler_params=pltpu.CompilerParams(
            dimension_semantics=("parallel", "parallel")
        ),
        interpret=interpret,
    )(x_pad, w_cat, b_cat)


def _sc_conv_body(n_dst, e_pad, n_ea, npart, bsz, interpret,
                  xl_hbm, xr_hbm, s_hbm, d_hbm, att_hbm, wea_hbm, ea_hbm,
                  out_hbm,
                  spm, ks, kd, keas, s_ch, d_ch, ea_chs, xl_b, xr_b,
                  out_b, widx, didx, sidx, att_v, wea_v, abuf, exv,
                  sem_r, sem_l, sem_s):
    q = n_dst // npart
    qpad = _round_up(q + 1, 128)
    rpt = qpad // 16
    sh = e_pad // 16
    nch = sh // CH
    npass = npart // 2
    bq = bsz // 16

    cid = lax.axis_index("c")
    sid = lax.axis_index("sub")
    r0 = sid * rpt
    shard_lo = sid * sh

    pltpu.sync_copy(att_hbm, att_v)
    if n_ea:
        pltpu.sync_copy(wea_hbm, wea_v)

    zv = jnp.zeros((16,), jnp.float32)

    def _zero_row(i, _):
        for j in range(NJ + 1):
            out_b[0][i, pl.ds(16 * j, 16)] = zv
        return 0

    pad_d = jnp.full((16,), q, jnp.int32)
    pad_s = jnp.zeros((16,), jnp.int32)
    pad_f = jnp.zeros((16,), jnp.float32)

    def _pass(p, _):
        part = npass * cid + p
        lo = part * q

        lax.fori_loop(0, bsz, _zero_row, 0)
        for c0 in range(0, rpt, bsz):
            sz = min(bsz, rpt - c0)
            pltpu.sync_copy(out_b[0].at[pl.ds(0, sz)],
                            spm.at[pl.ds(r0 + c0, sz)])
        plsc.subcore_barrier()

        lo_v = jnp.full((16,), lo, jnp.int32)

        def _issue(off, par, wr):
            for t in range(bq):
                sl = pl.ds(16 * t, 16)
                kdv = kd[pl.ds(off + 16 * t, 16)]
                widx[wr][sl] = kdv
                didx[par][sl] = kdv + lo_v
                sidx[par][sl] = ks[pl.ds(off + 16 * t, 16)]
            pltpu.async_copy(xr_hbm.at[didx[par]], xr_b[par], sem_r[par])
            pltpu.async_copy(xl_hbm.at[sidx[par]], xl_b[par], sem_l[par])

        def _gather_wait(par):
            pltpu.make_async_copy(xr_hbm.at[didx[par]], xr_b[par],
                                  sem_r[par]).wait()
            pltpu.make_async_copy(xl_hbm.at[sidx[par]], xl_b[par],
                                  sem_l[par]).wait()

        def _scatter_wait(wr, sp):
            pltpu.make_async_copy(out_b[sp], spm.at[widx[wr]],
                                  sem_s[sp]).wait()

        def _compute(off, par, wr):
            @plsc.parallel_loop(0, bsz, unroll=2)
            def _edge_a(i):
                bidx = jnp.full((16,), off + i, jnp.int32)
                eav = [plsc.load_gather(keas[t], [bidx]) for t in range(n_ea)]
                accs = [jnp.zeros((16,), jnp.float32) for _ in range(4)]
                for j in range(NJ):
                    sl = pl.ds(16 * j, 16)
                    m = xl_b[par][i, sl] + xr_b[par][i, sl]
                    for t in range(n_ea):
                        m = m + eav[t] * wea_v[t, j]
                    g = jnp.maximum(m, 0.2 * m)
                    accs[j % 4] = accs[j % 4] + g * att_v[j]
                abuf[pl.ds(16 * i, 16)] = (accs[0] + accs[1]) + (accs[2]
                                                                 + accs[3])

            iota16 = lax.iota(jnp.int32, 16) * 16

            @plsc.parallel_loop(0, bsz // 16, unroll=1)
            def _group_b(gidx):
                base = iota16 + gidx * 256
                tots = [plsc.load_gather(abuf, [base + j]) for j in range(4)]
                for j in range(4, 16):
                    tots[j % 4] = tots[j % 4] + plsc.load_gather(
                        abuf, [base + j])
                exv[pl.ds(16 * gidx, 16)] = jnp.exp((tots[0] + tots[1])
                                                    + (tots[2] + tots[3]))

            @plsc.parallel_loop(0, bsz, unroll=2)
            def _edge_c(i):
                ex = plsc.load_gather(exv, [jnp.full((16,), i, jnp.int32)])
                for j in range(NJ):
                    sl = pl.ds(16 * j, 16)
                    out_b[par][i, sl] = ex * xl_b[par][i, sl]
                out_b[par][i, pl.ds(D, 16)] = ex

            pltpu.async_copy(out_b[par], spm.at[widx[wr]], sem_s[par],
                             add=True)

        def _chunk(c, carry):
            gb, rem = carry
            c0 = shard_lo + c * CH
            pltpu.sync_copy(s_hbm.at[pl.ds(c0, CH)], s_ch)
            pltpu.sync_copy(d_hbm.at[pl.ds(c0, CH)], d_ch)
            for t in range(n_ea):
                pltpu.sync_copy(ea_hbm.at[t, pl.ds(c0, CH)], ea_chs[t])

            def _step(k, cnt):
                dv = d_ch[pl.ds(16 * k, 16)]
                sv = s_ch[pl.ds(16 * k, 16)]
                msk = (dv >= lo) & (dv < lo + q)
                plsc.store_compressed(ks.at[pl.ds(cnt, 16)], sv, mask=msk)
                plsc.store_compressed(kd.at[pl.ds(cnt, 16)], dv - lo,
                                      mask=msk)
                for t in range(n_ea):
                    plsc.store_compressed(keas[t].at[pl.ds(cnt, 16)],
                                          ea_chs[t][pl.ds(16 * k, 16)],
                                          mask=msk)
                return cnt + plsc.all_reduce_population_count(msk)[0]

            cnt = lax.fori_loop(0, CH // 16, _step, rem)

            def _finish(cnt):
                for t in range(bq):
                    ks[pl.ds(cnt + 16 * t, 16)] = pad_s
                    kd[pl.ds(cnt + 16 * t, 16)] = pad_d
                    for u in range(n_ea):
                        keas[u][pl.ds(cnt + 16 * t, 16)] = pad_f

            nb = cnt // bsz
            if True:
                is_last = c == nch - 1
                nb = jnp.where(is_last, (cnt + bsz - 1) // bsz, nb)

                @pl.when(jnp.logical_and(is_last, cnt % bsz != 0))
                def _():
                    _finish(cnt)

            def _pipe_step(bi, _):
                g = gb + bi

                def _one(k):
                    @pl.when(bi + 1 < nb)
                    def _():
                        _issue((bi + 1) * bsz, (k + 1) % 2, (k + 1) % 4)

                    @pl.when(g >= 2)
                    def _():
                        _scatter_wait((k + 2) % 4, k % 2)

                    _gather_wait(k % 2)
                    _compute(bi * bsz, k % 2, k)

                for k in range(4):
                    @pl.when(g % 4 == k)
                    def _(k=k):
                        _one(k)

                return 0

            @pl.when(nb >= 1)
            def _():
                for k in range(4):
                    @pl.when(gb % 4 == k)
                    def _(k=k):
                        _issue(0, k % 2, k)

            lax.fori_loop(0, nb, _pipe_step, 0)

            new_rem = cnt - nb * bsz

            @pl.when(jnp.logical_and(nb >= 1, new_rem > 0))
            def _():
                for t in range(bq):
                    sl = pl.ds(16 * t, 16)
                    ks[sl] = ks[pl.ds(nb * bsz + 16 * t, 16)]
                    kd[sl] = kd[pl.ds(nb * bsz + 16 * t, 16)]
                    for u in range(n_ea):
                        keas[u][sl] = keas[u][pl.ds(nb * bsz + 16 * t, 16)]

            return gb + nb, new_rem

        total, _rem = lax.fori_loop(0, nch, _chunk, (0, 0))

        for k in range(4):
            @pl.when(jnp.logical_and(total >= 1, (total - 1) % 4 == k))
            def _(k=k):
                _scatter_wait(k, k % 2)

            @pl.when(jnp.logical_and(total >= 2, (total - 2) % 4 == k))
            def _(k=k):
                _scatter_wait(k, k % 2)

        plsc.subcore_barrier()
        pltpu.sync_copy(spm.at[pl.ds(r0, rpt)],
                        out_hbm.at[pl.ds(cid * npass * qpad + p * qpad + r0,
                                         rpt)])
        return 0

    lax.fori_loop(0, npass, _pass, 0)


def _sc_conv(xl, xr, s_idx, d_idx, att, wea, ea_t, n_dst, interpret=False):
    e_pad = s_idx.shape[0]
    n_ea = 0 if ea_t is None else ea_t.shape[0]
    npart = NPART_BIG if n_dst >= 4096 else NPART
    bsz = 48 if n_ea else B
    q = n_dst // npart
    qpad = _round_up(q + 1, 128)
    cap = CH + 2 * B

    if ea_t is None:
        ea_t = jnp.zeros((1, e_pad), jnp.float32)
        wea = jnp.zeros((1, 16, 16), jnp.float32)

    mesh = plsc.VectorSubcoreMesh(core_axis_name="c", subcore_axis_name="sub",
                                  num_cores=2, num_subcores=16)
    scratch = [
        pltpu.VMEM_SHARED((qpad, D + 16), jnp.float32),
        pltpu.VMEM((cap,), jnp.int32),
        pltpu.VMEM((cap,), jnp.int32),
        [pltpu.VMEM((cap,), jnp.float32) for _ in range(n_ea)],
        pltpu.VMEM((CH,), jnp.int32),
        pltpu.VMEM((CH,), jnp.int32),
        [pltpu.VMEM((CH,), jnp.float32) for _ in range(n_ea)],
        [pltpu.VMEM((bsz, D), jnp.float32) for _ in range(2)],
        [pltpu.VMEM((bsz, D), jnp.float32) for _ in range(2)],
        [pltpu.VMEM((bsz, D + 16), jnp.float32) for _ in range(2)],
        [pltpu.VMEM((bsz,), jnp.int32) for _ in range(4)],
        [pltpu.VMEM((bsz,), jnp.int32) for _ in range(2)],
        [pltpu.VMEM((bsz,), jnp.int32) for _ in range(2)],
        pltpu.VMEM((NJ, 16), jnp.float32),
        pltpu.VMEM((max(n_ea, 1), NJ, 16), jnp.float32),
        pltpu.VMEM((bsz * 16,), jnp.float32),
        pltpu.VMEM((bsz,), jnp.float32),
        [pltpu.SemaphoreType.DMA for _ in range(2)],
        [pltpu.SemaphoreType.DMA for _ in range(2)],
        [pltpu.SemaphoreType.DMA for _ in range(2)],
    ]
    body = functools.partial(_sc_conv_body, n_dst, e_pad, n_ea, npart, bsz,
                             interpret)
    fn = pl.kernel(
        body,
        out_type=jax.ShapeDtypeStruct((npart * qpad, D + 16), jnp.float32),
        mesh=mesh,
        scratch_types=scratch,
        compiler_params=pltpu.CompilerParams(use_tc_tiling_on_sc=False,
                                             needs_layout_passes=False),
        interpret=interpret,
    )
    return fn(xl, xr, s_idx, d_idx, att, wea, ea_t)


def _make_combine_body(n_conv):
    def _body(*refs):
        x_ref = refs[0]
        b_ref = refs[1]
        o_ref = refs[-1]
        acc = x_ref[...] + b_ref[...]
        for c in range(n_conv):
            nd = refs[2 + c][...]
            acc = acc + nd[:, :D] / (nd[:, D:D + 1] + _EPS)
        o_ref[...] = acc
    return _body


def _combine(x_pad, bias_sum, numdens, interpret=False):
    n = x_pad.shape[0]
    br = 256
    n_conv = len(numdens)
    return pl.pallas_call(
        _make_combine_body(n_conv),
        grid=(n // br,),
        in_specs=[
            pl.BlockSpec((br, D), lambda i: (i, 0)),
            pl.BlockSpec((1, D), lambda i: (0, 0)),
        ] + [pl.BlockSpec((br, D + 16), lambda i: (i, 0)) for _ in numdens],
        out_specs=pl.BlockSpec((br, D), lambda i: (i, 0)),
        out_shape=jax.ShapeDtypeStruct((n, D), jnp.float32),
        compiler_params=pltpu.CompilerParams(
            dimension_semantics=("parallel",)
        ),
        interpret=interpret,
    )(x_pad, bias_sum, *numdens)


def _pad_rows(x, n):
    return jnp.pad(x, ((0, n - x.shape[0]), (0, 0)))


def _pad_edges(ei, ea, n_dst):
    e = ei.shape[1]
    e_pad = _round_up(e, 16 * 1024)
    s = jnp.pad(ei[0], (0, e_pad - e)).astype(jnp.int32)
    d = jnp.pad(ei[1], (0, e_pad - e),
                constant_values=2 * n_dst + 64).astype(jnp.int32)
    if ea is None:
        ea_t = None
    else:
        ea_t = jnp.pad(ea.T, ((0, 0), (0, e_pad - e))).astype(jnp.float32)
    return s, d, ea_t


def kernel(x_operation, x_machine, x_AGV, ei_op_pred, ei_op_succ,
           ei_m_processable, ei_m_processing, ea_m_processing,
           ei_m_waiting, ea_m_waiting, ei_a_pos_m, ei_a_tgt_m,
           ea_a_tgt_m, ei_a_pos_o, ei_a_tgt_o, params, interpret=False):
    p = params
    n_op, n_m, n_a = x_operation.shape[0], x_machine.shape[0], x_AGV.shape[0]
    n_op_pad, n_m_pad, n_a_pad = (_round_up(n_op, 256), _round_up(n_m, 256),
                                  _round_up(n_a, 256))
    x_op_pad = _pad_rows(x_operation, n_op_pad)
    x_m_pad = _pad_rows(x_machine, n_m_pad)
    x_a_pad = _pad_rows(x_AGV, n_a_pad)

    convs = [
        ("op_pred", p["op_pred"], "op", "op", ei_op_pred, False, None),
        ("op_succ", p["op_succ"], "op", "op", ei_op_succ, False, None),
        ("m_proc_f", p["m_processable"]["fwd"], "m", "op", ei_m_processable, False, None),
        ("m_proc_b", p["m_processable"]["bwd"], "op", "m", ei_m_processable, True, None),
        ("m_ing_f", p["m_processing"]["fwd"], "m", "op", ei_m_processing, False, ea_m_processing),
        ("m_ing_b", p["m_processing"]["bwd"], "op", "m", ei_m_processing, True, ea_m_processing),
        ("m_wait_f", p["m_waiting"]["fwd"], "m", "op", ei_m_waiting, False, ea_m_waiting),
        ("m_wait_b", p["m_waiting"]["bwd"], "op", "m", ei_m_waiting, True, ea_m_waiting),
        ("a_pos_m_f", p["a_pos_m"]["fwd"], "a", "m", ei_a_pos_m, False, None),
        ("a_pos_m_b", p["a_pos_m"]["bwd"], "m", "a", ei_a_pos_m, True, None),
        ("a_tgt_m_f", p["a_tgt_m"]["fwd"], "a", "m", ei_a_tgt_m, False, ea_a_tgt_m),
        ("a_tgt_m_b", p["a_tgt_m"]["bwd"], "m", "a", ei_a_tgt_m, True, ea_a_tgt_m),
        ("a_pos_o_f", p["a_pos_o"]["fwd"], "a", "op", ei_a_pos_o, False, None),
        ("a_pos_o_b", p["a_pos_o"]["bwd"], "op", "a", ei_a_pos_o, True, None),
        ("a_tgt_o_f", p["a_tgt_o"]["fwd"], "a", "op", ei_a_tgt_o, False, None),
        ("a_tgt_o_b", p["a_tgt_o"]["bwd"], "op", "a", ei_a_tgt_o, True, None),
    ]

    mats = {"op": [], "m": [], "a": []}
    for ci, (_, cp, sk, dk, _, _, _) in enumerate(convs):
        mats[sk].append((cp["Wl"], cp["bl"], ci, "l"))
        mats[dk].append((cp["Wr"], cp["br"], ci, "r"))

    xs = {"op": x_op_pad, "m": x_m_pad, "a": x_a_pad}
    tables = {}
    for kind, lst in mats.items():
        w_cat = jnp.concatenate([w for w, _, _, _ in lst], axis=1)
        b_cat = jnp.concatenate([b for _, b, _, _ in lst])[None, :]
        proj = _project(xs[kind], w_cat, b_cat, interpret=interpret)
        for t, (_, _, ci, side) in enumerate(lst):
            tables[(ci, side)] = proj[:, t * D:(t + 1) * D]

    n_dst_of = {"op": n_op, "m": n_m, "a": n_a}

    numdens = {"op": [], "m": [], "a": []}
    biases = {"op": [], "m": [], "a": []}
    for ci, (_, cp, sk, dk, ei, swap, ea) in enumerate(convs):
        n_dst = n_dst_of[dk]
        s_row, d_row = (ei[1], ei[0]) if swap else (ei[0], ei[1])
        s_idx, d_idx, ea_t = _pad_edges(jnp.stack([s_row, d_row]), ea, n_dst)
        att = cp["att"].reshape(NJ, 16)
        wea = (cp["We"].reshape(-1, NJ, 16) if ea is not None else None)
        nd = _sc_conv(tables[(ci, "l")], tables[(ci, "r")], s_idx, d_idx,
                      att, wea, ea_t, n_dst, interpret=interpret)
        npart = NPART_BIG if n_dst >= 4096 else NPART
        q = n_dst // npart
        qpad = _round_up(q + 1, 128)
        nd_real = jnp.concatenate(
            [nd[i * qpad:i * qpad + q] for i in range(npart)], axis=0)
        n_pad = {"op": n_op_pad, "m": n_m_pad, "a": n_a_pad}[dk]
        numdens[dk].append(jnp.pad(nd_real, ((0, n_pad - n_dst), (0, 0))))
        biases[dk].append(cp["bias"])

    res = {}
    for kind, x_pad in (("op", x_op_pad), ("m", x_m_pad), ("a", x_a_pad)):
        bias_sum = sum(biases[kind])[None, :]
        res[kind] = _combine(x_pad, bias_sum, numdens[kind], interpret=interpret)

    return jnp.concatenate(
        [res["op"][:n_op], res["m"][:n_m], res["a"][:n_a]], axis=0)
